# baseline (speedup 1.0000x reference)
import functools

import jax
import jax.numpy as jnp
import numpy as np
from jax import lax
from jax.experimental import pallas as pl
from jax.experimental.pallas import tpu as pltpu
from jax.experimental.pallas import tpu_sc as plsc

f32 = jnp.float32
i32 = jnp.int32

NN = 50000
NPAD = 50176
CH = 32
NCORE = 2
NSUB = 16
NW = NCORE * NSUB
CK = 128
NBLK = 1024
SCALE = float(1.0 / np.sqrt(CH))
NEG = -3.0e38

_SUB_SLICE = NPAD // NSUB


def _mesh():
  return plsc.VectorSubcoreMesh(core_axis_name="c", subcore_axis_name="s")


def _wid():
  return lax.axis_index("c") * NSUB + lax.axis_index("s")


def _iota16():
  return lax.iota(i32, 16)


def _splat(x):
  return jnp.full((16,), x, i32)


def _sc_posgather(pos_pad, src2, dst2, ep):
  ckp = 256
  epw = ep // NW
  nchunk = epw // ckp

  def body(pos_t, src_h, dst_h, pgs_o, pgd_o, srcb, dstb, rs, rd, semi, semg,
           semw):
    base = _wid() * nchunk * 2

    def chunk(ci, carry):
      bb = base + ci * 2
      b = bb * CK
      di = pltpu.async_copy(src_h.at[pl.ds(bb, 2)], srcb, semi)
      dj = pltpu.async_copy(dst_h.at[pl.ds(bb, 2)], dstb, semi)
      di.wait()
      dj.wait()
      g1 = pltpu.async_copy(pos_t.at[srcb.at[0]], rs.at[pl.ds(0, CK)], semg)
      g2 = pltpu.async_copy(pos_t.at[srcb.at[1]], rs.at[pl.ds(CK, CK)], semg)
      g3 = pltpu.async_copy(pos_t.at[dstb.at[0]], rd.at[pl.ds(0, CK)], semg)
      g4 = pltpu.async_copy(pos_t.at[dstb.at[1]], rd.at[pl.ds(CK, CK)], semg)
      g1.wait(); g2.wait(); g3.wait(); g4.wait()
      w1 = pltpu.async_copy(rs, pgs_o.at[pl.ds(b, ckp)], semw)
      w2 = pltpu.async_copy(rd, pgd_o.at[pl.ds(b, ckp)], semw)
      w1.wait(); w2.wait()
      return carry

    lax.fori_loop(0, nchunk, chunk, 0)

  fn = pl.kernel(
      body,
      out_type=(
          jax.ShapeDtypeStruct((ep, 8), f32),
          jax.ShapeDtypeStruct((ep, 8), f32),
      ),
      mesh=_mesh(),
      compiler_params=pltpu.CompilerParams(use_tc_tiling_on_sc=False, needs_layout_passes=False),
      scratch_types=[
          pltpu.VMEM((2, CK), i32),
          pltpu.VMEM((2, CK), i32),
          pltpu.VMEM((ckp, 8), f32),
          pltpu.VMEM((ckp, 8), f32),
          pltpu.SemaphoreType.DMA,
          pltpu.SemaphoreType.DMA,
          pltpu.SemaphoreType.DMA,
      ],
  )
  return fn(pos_pad, src2, dst2)


def _sc_pass_a(qtab, ktab, src2, dst2, ep):
  epw = ep // NW
  nchunk = epw // CK

  def body(q_t, k_t, src_h, dst_h, lo_o, mp_o, mtab, srcb, dstb, qr, kr,
           lstage, semi, semg, semw):
    w = _wid()
    base = w * nchunk
    it = _iota16()

    def initb(j, carry):
      plsc.store_scatter(mtab, [it + j * 16], jnp.full((16,), NEG, f32))
      return carry

    lax.fori_loop(0, NPAD // 16, initb, 0)

    def issue(ci, buf):
      i1_ = pltpu.async_copy(src_h.at[pl.ds(base + ci, 1)],
                             srcb.at[pl.ds(buf, 1)], semi)
      i2_ = pltpu.async_copy(dst_h.at[pl.ds(base + ci, 1)],
                             dstb.at[pl.ds(buf, 1)], semi)
      i1_.wait()
      i2_.wait()
      pltpu.async_copy(q_t.at[dstb.at[buf]], qr.at[pl.ds(buf * CK, CK)],
                       semg)
      pltpu.async_copy(k_t.at[srcb.at[buf]], kr.at[pl.ds(buf * CK, CK)],
                       semg)

    def drain(buf):
      pltpu.make_async_copy(q_t.at[dstb.at[buf]],
                            qr.at[pl.ds(buf * CK, CK)], semg).wait()
      pltpu.make_async_copy(k_t.at[srcb.at[buf]],
                            kr.at[pl.ds(buf * CK, CK)], semg).wait()

    issue(0, 0)

    def chunk(ci, carry):
      buf = lax.rem(ci, 2)

      @pl.when(ci + 1 < nchunk)
      def _():
        issue(ci + 1, 1 - buf)

      drain(buf)
      rbase = buf * CK

      def grp(j, carry2):
        lanes = j * 16 + it
        rows = jnp.full((16,), rbase, i32) + lanes
        dst16 = plsc.load_gather(dstb, [jnp.full((16,), buf, i32), lanes])
        acc = jnp.zeros((16,), f32)
        for c in range(CH):
          q16 = plsc.load_gather(qr, [rows, _splat(c)])
          k16 = plsc.load_gather(kr, [rows, _splat(c)])
          acc = acc + q16 * k16
        l16 = acc
        plsc.store_scatter(lstage, [lanes], l16)

        def wcond(cnt):
          t2 = plsc.load_gather(mtab, [dst16])
          return jnp.any(t2 < l16)

        def wbody(cnt):
          t2 = plsc.load_gather(mtab, [dst16])
          plsc.store_scatter(mtab, [dst16], jnp.maximum(t2, l16),
                             mask=t2 < l16)
          return cnt + 1

        lax.while_loop(wcond, wbody, 0)
        return carry2

      lax.fori_loop(0, CK // 16, grp, 0)
      pltpu.async_copy(lstage, lo_o.at[pl.ds((base + ci) * CK, CK)],
                       semw).wait()
      return carry

    lax.fori_loop(0, nchunk, chunk, 0)
    pltpu.sync_copy(mtab, mp_o.at[w])

  fn = pl.kernel(
      body,
      out_type=(
          jax.ShapeDtypeStruct((ep,), f32),
          jax.ShapeDtypeStruct((NW, NPAD), f32),
      ),
      mesh=_mesh(),
      compiler_params=pltpu.CompilerParams(use_tc_tiling_on_sc=False, needs_layout_passes=False),
      scratch_types=[
          pltpu.VMEM((NPAD,), f32),
          pltpu.VMEM((2, CK), i32),
          pltpu.VMEM((2, CK), i32),
          pltpu.VMEM((2 * CK, CH), f32),
          pltpu.VMEM((2 * CK, CH), f32),
          pltpu.VMEM((CK,), f32),
          pltpu.SemaphoreType.DMA,
          pltpu.SemaphoreType.DMA,
          pltpu.SemaphoreType.DMA,
      ],
  )
  return fn(qtab, ktab, src2, dst2)


def _sc_pass_b(lo, dst2, mvec, zeros32, ep):
  ckb = 512
  epw = ep // NW
  nchunk = epw // ckb

  def body(lo_h, dst_h, m_h, z_h, ex_o, dp_o, mtab, dstb, lbuf, exst, exw,
           spden, semi, semw, sems):
    core = lax.axis_index("c")
    sub = lax.axis_index("s")
    base = (core * NSUB + sub) * nchunk * 4
    it = _iota16()

    pltpu.sync_copy(m_h, mtab)
    pltpu.sync_copy(z_h.at[pl.ds(sub * _SUB_SLICE, _SUB_SLICE), pl.ds(0, 8)],
                    spden.at[pl.ds(sub * _SUB_SLICE, _SUB_SLICE)])
    pltpu.sync_copy(z_h.at[pl.ds(0, ckb), pl.ds(0, 8)], exst)
    plsc.subcore_barrier()

    def chunk(ci, carry):
      bb = base + ci * 4
      b = bb * CK
      di = pltpu.async_copy(dst_h.at[pl.ds(bb, 4)], dstb, semi)
      dl = pltpu.async_copy(lo_h.at[pl.ds(b, ckb)], lbuf, semi)
      di.wait()
      dl.wait()

      def grp(j, carry2):
        rows = it + j * 16
        dst16 = plsc.load_gather(dstb, [jnp.full((16,), j >> 3, i32),
                                        (j & 7) * 16 + it])
        l16 = plsc.load_gather(lbuf, [rows])
        m16 = plsc.load_gather(mtab, [dst16])
        e16 = jnp.exp(l16 - m16)
        plsc.store_scatter(exst, [rows, _splat(0)], e16)
        plsc.store_scatter(exw, [rows], e16)
        return carry2

      lax.fori_loop(0, ckb // 16, grp, 0)
      pltpu.async_copy(exw, ex_o.at[pl.ds(b, ckb)], semw).wait()
      s1 = pltpu.async_copy(exst.at[pl.ds(0, CK)], spden.at[dstb.at[0]],
                            sems, add=True)
      s2 = pltpu.async_copy(exst.at[pl.ds(CK, CK)], spden.at[dstb.at[1]],
                            sems, add=True)
      s3 = pltpu.async_copy(exst.at[pl.ds(2 * CK, CK)], spden.at[dstb.at[2]],
                            sems, add=True)
      s4 = pltpu.async_copy(exst.at[pl.ds(3 * CK, CK)], spden.at[dstb.at[3]],
                            sems, add=True)
      s1.wait(); s2.wait(); s3.wait(); s4.wait()
      return carry

    lax.fori_loop(0, nchunk, chunk, 0)
    plsc.subcore_barrier()
    pltpu.sync_copy(spden.at[pl.ds(sub * _SUB_SLICE, _SUB_SLICE)],
                    dp_o.at[core, pl.ds(sub * _SUB_SLICE, _SUB_SLICE)])

  fn = pl.kernel(
      body,
      out_type=(
          jax.ShapeDtypeStruct((ep,), f32),
          jax.ShapeDtypeStruct((NCORE, NPAD, 8), f32),
      ),
      mesh=_mesh(),
      compiler_params=pltpu.CompilerParams(use_tc_tiling_on_sc=False, needs_layout_passes=False),
      scratch_types=[
          pltpu.VMEM((NPAD,), f32),
          pltpu.VMEM((4, CK), i32),
          pltpu.VMEM((ckb,), f32),
          pltpu.VMEM((ckb, 8), f32),
          pltpu.VMEM((ckb,), f32),
          pltpu.VMEM_SHARED((NPAD, 8), f32),
          pltpu.SemaphoreType.DMA,
          pltpu.SemaphoreType.DMA,
          pltpu.SemaphoreType.DMA,
      ],
  )
  return fn(lo, dst2, mvec, zeros32)


def _sc_pass_c(v0, v1a, v1b, v1c, es0f, es1f, eu0f, eu1f, eu2f, src2,
               dst2, zeros32, ep):
  epw = ep // NW
  nchunk = epw // CK
  fpc = CK * CH

  def body(v0_t, va_t, vb_t, vc_t, e0_h, e1_h, u0_h, u1_h, u2_h, src_h,
           dst_h, z_h, agg_o, srcb, dstb, vr, sab, sbb, spagg, semi, semg,
           seml, sems):
    core = lax.axis_index("c")
    sub = lax.axis_index("s")
    base = (core * NSUB + sub) * nchunk
    it = _iota16()

    def one_group(g, v_t, sa_h, sb_h):
      with_u = sb_h is not None
      pltpu.sync_copy(z_h.at[pl.ds(sub * _SUB_SLICE, _SUB_SLICE)],
                      spagg.at[pl.ds(sub * _SUB_SLICE, _SUB_SLICE)])
      plsc.subcore_barrier()

      def issue(ci, buf):
        i1_ = pltpu.async_copy(src_h.at[pl.ds(base + ci, 1)],
                               srcb.at[pl.ds(buf, 1)], semi)
        i2_ = pltpu.async_copy(dst_h.at[pl.ds(base + ci, 1)],
                               dstb.at[pl.ds(buf, 1)], semi)
        pltpu.async_copy(sa_h.at[pl.ds((base + ci) * fpc, fpc)],
                         sab.at[pl.ds(buf * fpc, fpc)], seml)
        if with_u:
          pltpu.async_copy(sb_h.at[pl.ds((base + ci) * fpc, fpc)],
                           sbb.at[pl.ds(buf * fpc, fpc)], seml)
        i1_.wait()
        i2_.wait()
        pltpu.async_copy(v_t.at[srcb.at[buf]], vr.at[pl.ds(buf * CK, CK)],
                         semg)

      def drain_loads(buf):
        pltpu.make_async_copy(sa_h.at[pl.ds(0, fpc)],
                              sab.at[pl.ds(buf * fpc, fpc)], seml).wait()
        if with_u:
          pltpu.make_async_copy(sb_h.at[pl.ds(0, fpc)],
                                sbb.at[pl.ds(buf * fpc, fpc)], seml).wait()
        pltpu.make_async_copy(v_t.at[srcb.at[buf]],
                              vr.at[pl.ds(buf * CK, CK)], semg).wait()

      issue(0, 0)

      def chunk(ci, carry):
        buf = lax.rem(ci, 2)
        nbuf = 1 - buf

        @pl.when(ci > 0)
        def _():
          pltpu.make_async_copy(vr.at[pl.ds(nbuf * CK, CK)],
                                spagg.at[dstb.at[nbuf]], sems).wait()

        @pl.when(ci + 1 < nchunk)
        def _():
          issue(ci + 1, nbuf)

        drain_loads(buf)
        rbase = buf * CK
        fbase = buf * fpc

        def mgrp(jo, carry2):
          gb = jo * 8
          for jj in range(8):
            g2 = gb + jj
            erow = jnp.full((16,), rbase, i32) + _splat(g2 >> 1)
            cols = it + (g2 & 1) * 16
            fidx = it + fbase + g2 * 16
            v16 = plsc.load_gather(vr, [erow, cols])
            s16 = plsc.load_gather(sab, [fidx])
            if with_u:
              s2_16 = plsc.load_gather(sbb, [fidx])
              msg = v16 * s16 + s2_16
            else:
              msg = v16 * s16
            plsc.store_scatter(vr, [erow, cols], msg)
          return carry2

        lax.fori_loop(0, (2 * CK) // 8, mgrp, 0)
        pltpu.async_copy(vr.at[pl.ds(rbase, CK)], spagg.at[dstb.at[buf]],
                         sems, add=True)
        return carry

      lax.fori_loop(0, nchunk, chunk, 0)
      lastbuf = lax.rem(nchunk - 1, 2)
      pltpu.make_async_copy(vr.at[pl.ds(lastbuf * CK, CK)],
                            spagg.at[dstb.at[lastbuf]], sems).wait()
      plsc.subcore_barrier()
      pltpu.sync_copy(spagg.at[pl.ds(sub * _SUB_SLICE, _SUB_SLICE)],
                      agg_o.at[core, g, pl.ds(sub * _SUB_SLICE, _SUB_SLICE)])

    one_group(0, v0_t, e0_h, None)
    one_group(1, va_t, e1_h, u0_h)
    one_group(2, vb_t, e1_h, u1_h)
    one_group(3, vc_t, e1_h, u2_h)

  scratch = [
      pltpu.VMEM((2, CK), i32),
      pltpu.VMEM((2, CK), i32),
      pltpu.VMEM((2 * CK, CH), f32),
      pltpu.VMEM((2 * CK * CH,), f32),
      pltpu.VMEM((2 * CK * CH,), f32),
      pltpu.VMEM_SHARED((NPAD, CH), f32),
      pltpu.SemaphoreType.DMA,
      pltpu.SemaphoreType.DMA,
      pltpu.SemaphoreType.DMA,
      pltpu.SemaphoreType.DMA,
  ]
  fn = pl.kernel(
      body,
      out_type=jax.ShapeDtypeStruct((NCORE, 4, NPAD, CH), f32),
      mesh=_mesh(),
      compiler_params=pltpu.CompilerParams(use_tc_tiling_on_sc=False, needs_layout_passes=False),
      scratch_types=scratch,
  )
  return fn(v0, v1a, v1b, v1c, es0f, es1f, eu0f, eu1f, eu2f, src2, dst2,
            zeros32)


def _tc_prep(pgs, pgd, ep):
  def body(pgs_r, pgd_r, ud_r):
    rel = pgs_r[...] - pgd_r[...]
    d2 = jnp.sum(rel * rel, axis=1, keepdims=True)
    dist = jnp.sqrt(d2)
    u = rel[:, :3] / (dist + 1e-6)
    zpad = jnp.zeros((rel.shape[0], 4), f32)
    ud_r[...] = jnp.concatenate([u, dist, zpad], axis=1)

  return pl.pallas_call(
      body,
      grid=(ep // NBLK,),
      in_specs=[
          pl.BlockSpec((NBLK, 8), lambda i: (i, 0)),
          pl.BlockSpec((NBLK, 8), lambda i: (i, 0)),
      ],
      out_specs=pl.BlockSpec((NBLK, 8), lambda i: (i, 0)),
      out_shape=jax.ShapeDtypeStruct((ep, 8), f32),
  )(pgs, pgd)


def _tc_nodeproj(h0, h1, wq, wk, wv0, wv1, wself):
  def body(h0_r, h1_r, wq_r, wk_r, wv0_r, wv1_r, ws_r, q_r, k_r, v0_r,
           v1a_r, v1b_r, v1c_r, sc_r):
    h0b = h0_r[...]
    q_r[...] = jnp.dot(h0b, wq_r[...]) * SCALE
    k_r[...] = jnp.dot(h0b, wk_r[...])
    v0_r[...] = jnp.dot(h0b, wv0_r[...])
    sc_r[...] = jnp.dot(h0b, ws_r[...])
    h1b = h1_r[...]
    wv1b = wv1_r[...]
    v1a_r[...] = jnp.dot(h1b[0], wv1b)
    v1b_r[...] = jnp.dot(h1b[1], wv1b)
    v1c_r[...] = jnp.dot(h1b[2], wv1b)

  wspec = pl.BlockSpec((CH, CH), lambda i: (0, 0))
  nspec = pl.BlockSpec((NBLK, CH), lambda i: (i, 0))
  tspec = pl.BlockSpec((3, NBLK, CH), lambda i: (0, i, 0))
  return pl.pallas_call(
      body,
      grid=(NPAD // NBLK,),
      in_specs=[nspec, tspec, wspec, wspec, wspec, wspec, wspec],
      out_specs=[nspec] * 3 + [nspec] * 3 + [nspec],
      out_shape=[jax.ShapeDtypeStruct((NPAD, CH), f32)] * 7,
  )(h0, h1, wq, wk, wv0, wv1, wself)


def _tc_radial(ud, ea, wr1d, wr1e, br1, wr2a, wr2b, wr2c, ep):
  def body(ud_r, ea_r, w1d_r, w1e_r, b1_r, w2a_r, w2b_r, w2c_r, s0_r, s1_r,
           s2_r):
    dist = ud_r[:, 3:4]
    h = jnp.maximum(
        jnp.dot(ea_r[...], w1e_r[...]) + dist * w1d_r[...] + b1_r[...], 0.0)
    s0_r[...] = jnp.dot(h, w2a_r[...])
    s1_r[...] = jnp.dot(h, w2b_r[...])
    s2_r[...] = jnp.dot(h, w2c_r[...])

  espec = pl.BlockSpec((NBLK, CH), lambda i: (i, 0))
  wspec = pl.BlockSpec((CH, CH), lambda i: (0, 0))
  rspec = pl.BlockSpec((1, CH), lambda i: (0, 0))
  return pl.pallas_call(
      body,
      grid=(ep // NBLK,),
      in_specs=[
          pl.BlockSpec((NBLK, 8), lambda i: (i, 0)), espec, rspec, wspec,
          rspec, wspec, wspec, wspec
      ],
      out_specs=[espec, espec, espec],
      out_shape=[
          jax.ShapeDtypeStruct((ep, CH), f32),
          jax.ShapeDtypeStruct((ep, CH), f32),
          jax.ShapeDtypeStruct((ep, CH), f32),
      ],
  )(ud, ea, wr1d, wr1e, br1, wr2a, wr2b, wr2c)


def _tc_premult(ex, ud, s0, s1, s2, ep):
  def body(ex_r, ud_r, s0_r, s1_r, s2_r, es0_r, es1_r, eu0_r, eu1_r, eu2_r):
    exb = ex_r[...][:, None]
    es0_r[...] = exb * s0_r[...]
    es1_r[...] = exb * s1_r[...]
    es2 = exb * s2_r[...]
    udb = ud_r[...]
    eu0_r[...] = udb[:, 0:1] * es2
    eu1_r[...] = udb[:, 1:2] * es2
    eu2_r[...] = udb[:, 2:3] * es2

  espec = pl.BlockSpec((NBLK, CH), lambda i: (i, 0))
  eshape = jax.ShapeDtypeStruct((ep, CH), f32)
  return pl.pallas_call(
      body,
      grid=(ep // NBLK,),
      in_specs=[
          pl.BlockSpec((NBLK,), lambda i: (i,)),
          pl.BlockSpec((NBLK, 8), lambda i: (i, 0)),
          espec, espec, espec,
      ],
      out_specs=[espec] * 5,
      out_shape=[eshape] * 5,
  )(ex, ud, s0, s1, s2)


def _tc_combine_max(mpart):
  def body(mp_r, m_r):
    m_r[...] = jnp.max(mp_r[...], axis=0)

  return pl.pallas_call(
      body,
      grid=(NPAD // NBLK,),
      in_specs=[pl.BlockSpec((NW, NBLK), lambda i: (0, i))],
      out_specs=pl.BlockSpec((NBLK,), lambda i: (i,)),
      out_shape=jax.ShapeDtypeStruct((NPAD,), f32),
  )(mpart)


def _tc_combine_den(denp):
  def body(dp_r, d_r):
    d_r[...] = dp_r[0, :, 0] + dp_r[1, :, 0]

  return pl.pallas_call(
      body,
      grid=(NPAD // NBLK,),
      in_specs=[pl.BlockSpec((NCORE, NBLK, 8), lambda i: (0, i, 0))],
      out_specs=pl.BlockSpec((NBLK,), lambda i: (i,)),
      out_shape=jax.ShapeDtypeStruct((NPAD,), f32),
  )(denp)


def _tc_update(aggp, selfc, h1, den):
  def body(ap_r, sc_r, h1_r, den_r, h0o_r, h1o_r):
    rden = 1.0 / (den_r[...] + 1e-9)
    ap = ap_r[...]
    h0o_r[...] = jnp.maximum((ap[0, 0] + ap[1, 0]) * rden[:, None]
                             + sc_r[...], 0.0)
    a1 = ap[0, 1:4] + ap[1, 1:4]
    h1o_r[...] = h1_r[...] + a1 * rden[None, :, None]

  nspec = pl.BlockSpec((NBLK, CH), lambda i: (i, 0))
  tspec = pl.BlockSpec((3, NBLK, CH), lambda i: (0, i, 0))
  return pl.pallas_call(
      body,
      grid=(NPAD // NBLK,),
      in_specs=[
          pl.BlockSpec((NCORE, 4, NBLK, CH), lambda i: (0, 0, i, 0)),
          nspec,
          tspec,
          pl.BlockSpec((NBLK,), lambda i: (i,)),
      ],
      out_specs=[nspec, tspec],
      out_shape=[
          jax.ShapeDtypeStruct((NPAD, CH), f32),
          jax.ShapeDtypeStruct((3, NPAD, CH), f32),
      ],
  )(aggp, selfc, h1, den)


def _tc_heads(h0, h1, wout0, wout1, wwo, bwo, wwb, wwc1, bwc1, wwc2, wc2):
  def body(h0_r, h1_r, wo0_r, wo1_r, wwo_r, bwo_r, wwb_r, wc1_r, bc1_r,
           wc2_r, wcs_r, out_r):
    hs0 = jnp.dot(h0_r[...], wo0_r[...])
    h1b = h1_r[...]
    hs1 = jnp.einsum("dnc,co->dno", h1b, wo1_r[...],
                     preferred_element_type=f32)
    wo = jnp.tanh(jnp.dot(hs0, wwo_r[...]) + bwo_r[...])
    wb = jnp.tanh(jnp.dot(hs0, wwb_r[...]))
    wcmid = jnp.dot(hs0, wc1_r[...]) + bc1_r[...]
    wc = jnp.maximum(jnp.dot(wcmid, wc2_r[...]), 0.0)
    cs = jnp.dot(hs0, wcs_r[...])
    nb = hs0.shape[0]
    hs1flat = jnp.concatenate([hs1[0], hs1[1], hs1[2]], axis=1)
    out_r[...] = jnp.concatenate(
        [wo, wb, wc, cs, hs1flat, hs0,
         jnp.zeros((nb, 128 - 117), f32)], axis=1)

  nspec = pl.BlockSpec((NBLK, CH), lambda i: (i, 0))
  tspec = pl.BlockSpec((3, NBLK, CH), lambda i: (0, i, 0))

  def w(shape):
    return pl.BlockSpec(shape, lambda i: tuple(0 for _ in shape))

  return pl.pallas_call(
      body,
      grid=(NPAD // NBLK,),
      in_specs=[
          nspec, tspec,
          w((CH, 32)), w((CH, 8)), w((32, 8)), w((1, 8)), w((32, 8)),
          w((32, 32)), w((1, 32)), w((32, 15)), w((32, 30)),
      ],
      out_specs=pl.BlockSpec((NBLK, 128), lambda i: (i, 0)),
      out_shape=jax.ShapeDtypeStruct((NPAD, 128), f32),
  )(h0, h1, wout0, wout1, wwo, bwo, wwb, wwc1, bwc1, wwc2, wc2)


def kernel(x0, edge_index, edge_attr, pos, params):
  p = params
  e_real = edge_index.shape[1]
  ep = ((e_real + NW * CK - 1) // (NW * CK)) * (NW * CK)

  src = jnp.pad(edge_index[0].astype(i32), (0, ep - e_real))
  dst = jnp.pad(edge_index[1].astype(i32), (0, ep - e_real),
                constant_values=NN)
  ea = jnp.pad(edge_attr[:, :, 0], ((0, ep - e_real), (0, 0)))
  pos_pad = jnp.pad(pos, ((0, NPAD - NN), (0, 5)))
  h0 = jnp.pad(x0[:, :, 0], ((0, NPAD - NN), (0, 0)))
  h1 = jnp.zeros((3, NPAD, CH), f32)
  zeros32 = jnp.zeros((NPAD, CH), f32)

  src2 = src.reshape(-1, CK)
  dst2 = dst.reshape(-1, CK)
  pgs, pgd = _sc_posgather(pos_pad, src2, dst2, ep)
  ud = _tc_prep(pgs, pgd, ep)

  for l in range(2):
    q, k, v0, v1a, v1b, v1c, selfc = _tc_nodeproj(
        h0, h1, p["Wq%d" % l], p["Wk%d" % l], p["Wv0%d" % l],
        p["Wv1%d" % l], p["Wself%d" % l])
    wr1 = p["Wr1%d" % l]
    wr2 = p["Wr2%d" % l]
    s0, s1, s2 = _tc_radial(
        ud, ea, wr1[0:1], wr1[1:33], p["br1%d" % l].reshape(1, CH),
        wr2[:, 0:CH], wr2[:, CH:2 * CH], wr2[:, 2 * CH:3 * CH], ep)
    lo, mpart = _sc_pass_a(q, k, src2, dst2, ep)
    mvec = _tc_combine_max(mpart)
    ex, denp = _sc_pass_b(lo, dst2, mvec, zeros32, ep)
    den = _tc_combine_den(denp)
    es0, es1, eu0, eu1, eu2 = _tc_premult(ex, ud, s0, s1, s2, ep)
    aggp = _sc_pass_c(v0, v1a, v1b, v1c, es0.reshape(-1), es1.reshape(-1),
                      eu0.reshape(-1), eu1.reshape(-1), eu2.reshape(-1),
                      src2, dst2, zeros32, ep)
    h0, h1 = _tc_update(aggp, selfc, h1, den)

  wc_mat = jnp.transpose(p["Wc"], (1, 0, 2)).reshape(CH, 30)
  pack = _tc_heads(h0, h1, p["Wout0"], p["Wout1"], p["Wwo"],
                   p["bwo"].reshape(1, 8), p["Wwb"], p["Wwc1"],
                   p["bwc1"].reshape(1, CH), p["Wwc2"], wc_mat)

  n = NN
  wo = pack[:n, 0:8]
  wb = pack[:n, 8:16]
  wc = pack[:n, 16:31]
  cs = pack[:n, 31:61].reshape(n, 15, 2).transpose(1, 0, 2)
  hs1 = pack[:n, 61:85].reshape(n, 3, 8).transpose(0, 2, 1)
  hs0 = pack[:n, 85:117]
  return (wo, wb, wc, cs, hs0, hs1)


# speedup vs baseline: 1.1537x; 1.1537x over previous
import functools

import jax
import jax.numpy as jnp
import numpy as np
from jax import lax
from jax.experimental import pallas as pl
from jax.experimental.pallas import tpu as pltpu
from jax.experimental.pallas import tpu_sc as plsc

f32 = jnp.float32
i32 = jnp.int32

NN = 50000
NPAD = 50176
CH = 32
NCORE = 2
NSUB = 16
NW = NCORE * NSUB
CK = 128
NBLK = 1024
SCALE = float(1.0 / np.sqrt(CH))
NEG = -3.0e38

_SUB_SLICE = NPAD // NSUB


def _mesh():
  return plsc.VectorSubcoreMesh(core_axis_name="c", subcore_axis_name="s")


def _wid():
  return lax.axis_index("c") * NSUB + lax.axis_index("s")


def _iota16():
  return lax.iota(i32, 16)


def _splat(x):
  return jnp.full((16,), x, i32)


def _sc_posgather(pos_pad, src2, dst2, ep):
  ckp = 256
  epw = ep // NW
  nchunk = epw // ckp

  def body(pos_t, src_h, dst_h, pgs_o, pgd_o, srcb, dstb, rs, rd, semi, semg,
           semw):
    base = _wid() * nchunk * 2

    def chunk(ci, carry):
      bb = base + ci * 2
      b = bb * CK
      di = pltpu.async_copy(src_h.at[pl.ds(bb, 2)], srcb, semi)
      dj = pltpu.async_copy(dst_h.at[pl.ds(bb, 2)], dstb, semi)
      di.wait()
      dj.wait()
      g1 = pltpu.async_copy(pos_t.at[srcb.at[0]], rs.at[pl.ds(0, CK)], semg)
      g2 = pltpu.async_copy(pos_t.at[srcb.at[1]], rs.at[pl.ds(CK, CK)], semg)
      g3 = pltpu.async_copy(pos_t.at[dstb.at[0]], rd.at[pl.ds(0, CK)], semg)
      g4 = pltpu.async_copy(pos_t.at[dstb.at[1]], rd.at[pl.ds(CK, CK)], semg)
      g1.wait(); g2.wait(); g3.wait(); g4.wait()
      w1 = pltpu.async_copy(rs, pgs_o.at[pl.ds(b, ckp)], semw)
      w2 = pltpu.async_copy(rd, pgd_o.at[pl.ds(b, ckp)], semw)
      w1.wait(); w2.wait()
      return carry

    lax.fori_loop(0, nchunk, chunk, 0)

  fn = pl.kernel(
      body,
      out_type=(
          jax.ShapeDtypeStruct((ep, 8), f32),
          jax.ShapeDtypeStruct((ep, 8), f32),
      ),
      mesh=_mesh(),
      compiler_params=pltpu.CompilerParams(use_tc_tiling_on_sc=False, needs_layout_passes=False),
      scratch_types=[
          pltpu.VMEM((2, CK), i32),
          pltpu.VMEM((2, CK), i32),
          pltpu.VMEM((ckp, 8), f32),
          pltpu.VMEM((ckp, 8), f32),
          pltpu.SemaphoreType.DMA,
          pltpu.SemaphoreType.DMA,
          pltpu.SemaphoreType.DMA,
      ],
  )
  return fn(pos_pad, src2, dst2)


def _sc_pass_a(qtab, ktab, src2, dst2, ep):
  epw = ep // NW
  nchunk = epw // CK

  def body(q_t, k_t, src_h, dst_h, lo_o, mp_o, mtab, srcb, dstb, qr, kr,
           lstage, semi, semg, semw):
    w = _wid()
    base = w * nchunk
    it = _iota16()

    def initb(j, carry):
      plsc.store_scatter(mtab, [it + j * 16], jnp.full((16,), NEG, f32))
      return carry

    lax.fori_loop(0, NPAD // 16, initb, 0)

    def issue(ci, buf):
      i1_ = pltpu.async_copy(src_h.at[pl.ds(base + ci, 1)],
                             srcb.at[pl.ds(buf, 1)], semi)
      i2_ = pltpu.async_copy(dst_h.at[pl.ds(base + ci, 1)],
                             dstb.at[pl.ds(buf, 1)], semi)
      i1_.wait()
      i2_.wait()
      pltpu.async_copy(q_t.at[dstb.at[buf]], qr.at[pl.ds(buf * CK, CK)],
                       semg)
      pltpu.async_copy(k_t.at[srcb.at[buf]], kr.at[pl.ds(buf * CK, CK)],
                       semg)

    def drain(buf):
      pltpu.make_async_copy(q_t.at[dstb.at[buf]],
                            qr.at[pl.ds(buf * CK, CK)], semg).wait()
      pltpu.make_async_copy(k_t.at[srcb.at[buf]],
                            kr.at[pl.ds(buf * CK, CK)], semg).wait()

    issue(0, 0)

    def chunk(ci, carry):
      buf = lax.rem(ci, 2)

      @pl.when(ci + 1 < nchunk)
      def _():
        issue(ci + 1, 1 - buf)

      drain(buf)
      rbase = buf * CK

      def grp(j, carry2):
        lanes = j * 16 + it
        rows = jnp.full((16,), rbase, i32) + lanes
        dst16 = plsc.load_gather(dstb, [jnp.full((16,), buf, i32), lanes])
        acc = jnp.zeros((16,), f32)
        for c in range(CH):
          q16 = plsc.load_gather(qr, [rows, _splat(c)])
          k16 = plsc.load_gather(kr, [rows, _splat(c)])
          acc = acc + q16 * k16
        l16 = acc
        plsc.store_scatter(lstage, [lanes], l16)

        def wcond(cnt):
          t2 = plsc.load_gather(mtab, [dst16])
          return jnp.any(t2 < l16)

        def wbody(cnt):
          t2 = plsc.load_gather(mtab, [dst16])
          plsc.store_scatter(mtab, [dst16], jnp.maximum(t2, l16),
                             mask=t2 < l16)
          return cnt + 1

        lax.while_loop(wcond, wbody, 0)
        return carry2

      lax.fori_loop(0, CK // 16, grp, 0)
      pltpu.async_copy(lstage, lo_o.at[pl.ds((base + ci) * CK, CK)],
                       semw).wait()
      return carry

    lax.fori_loop(0, nchunk, chunk, 0)
    pltpu.sync_copy(mtab, mp_o.at[w])

  fn = pl.kernel(
      body,
      out_type=(
          jax.ShapeDtypeStruct((ep,), f32),
          jax.ShapeDtypeStruct((NW, NPAD), f32),
      ),
      mesh=_mesh(),
      compiler_params=pltpu.CompilerParams(use_tc_tiling_on_sc=False, needs_layout_passes=False),
      scratch_types=[
          pltpu.VMEM((NPAD,), f32),
          pltpu.VMEM((2, CK), i32),
          pltpu.VMEM((2, CK), i32),
          pltpu.VMEM((2 * CK, CH), f32),
          pltpu.VMEM((2 * CK, CH), f32),
          pltpu.VMEM((CK,), f32),
          pltpu.SemaphoreType.DMA,
          pltpu.SemaphoreType.DMA,
          pltpu.SemaphoreType.DMA,
      ],
  )
  return fn(qtab, ktab, src2, dst2)


def _sc_pass_b(lo, dst2, mvec, zeros32, ep):
  ckb = 512
  epw = ep // NW
  nchunk = epw // ckb

  def body(lo_h, dst_h, m_h, z_h, ex_o, dp_o, mtab, dstb, lbuf, exst, exw,
           spden, semi, semw, sems):
    core = lax.axis_index("c")
    sub = lax.axis_index("s")
    base = (core * NSUB + sub) * nchunk * 4
    it = _iota16()

    pltpu.sync_copy(m_h, mtab)
    pltpu.sync_copy(z_h.at[pl.ds(sub * _SUB_SLICE, _SUB_SLICE), pl.ds(0, 8)],
                    spden.at[pl.ds(sub * _SUB_SLICE, _SUB_SLICE)])
    pltpu.sync_copy(z_h.at[pl.ds(0, ckb), pl.ds(0, 8)], exst)
    plsc.subcore_barrier()

    def chunk(ci, carry):
      bb = base + ci * 4
      b = bb * CK
      di = pltpu.async_copy(dst_h.at[pl.ds(bb, 4)], dstb, semi)
      dl = pltpu.async_copy(lo_h.at[pl.ds(b, ckb)], lbuf, semi)
      di.wait()
      dl.wait()

      def grp(j, carry2):
        rows = it + j * 16
        dst16 = plsc.load_gather(dstb, [jnp.full((16,), j >> 3, i32),
                                        (j & 7) * 16 + it])
        l16 = plsc.load_gather(lbuf, [rows])
        m16 = plsc.load_gather(mtab, [dst16])
        e16 = jnp.exp(l16 - m16)
        plsc.store_scatter(exst, [rows, _splat(0)], e16)
        plsc.store_scatter(exw, [rows], e16)
        return carry2

      lax.fori_loop(0, ckb // 16, grp, 0)
      pltpu.async_copy(exw, ex_o.at[pl.ds(b, ckb)], semw).wait()
      s1 = pltpu.async_copy(exst.at[pl.ds(0, CK)], spden.at[dstb.at[0]],
                            sems, add=True)
      s2 = pltpu.async_copy(exst.at[pl.ds(CK, CK)], spden.at[dstb.at[1]],
                            sems, add=True)
      s3 = pltpu.async_copy(exst.at[pl.ds(2 * CK, CK)], spden.at[dstb.at[2]],
                            sems, add=True)
      s4 = pltpu.async_copy(exst.at[pl.ds(3 * CK, CK)], spden.at[dstb.at[3]],
                            sems, add=True)
      s1.wait(); s2.wait(); s3.wait(); s4.wait()
      return carry

    lax.fori_loop(0, nchunk, chunk, 0)
    plsc.subcore_barrier()
    pltpu.sync_copy(spden.at[pl.ds(sub * _SUB_SLICE, _SUB_SLICE)],
                    dp_o.at[core, pl.ds(sub * _SUB_SLICE, _SUB_SLICE)])

  fn = pl.kernel(
      body,
      out_type=(
          jax.ShapeDtypeStruct((ep,), f32),
          jax.ShapeDtypeStruct((NCORE, NPAD, 8), f32),
      ),
      mesh=_mesh(),
      compiler_params=pltpu.CompilerParams(use_tc_tiling_on_sc=False, needs_layout_passes=False),
      scratch_types=[
          pltpu.VMEM((NPAD,), f32),
          pltpu.VMEM((4, CK), i32),
          pltpu.VMEM((ckb,), f32),
          pltpu.VMEM((ckb, 8), f32),
          pltpu.VMEM((ckb,), f32),
          pltpu.VMEM_SHARED((NPAD, 8), f32),
          pltpu.SemaphoreType.DMA,
          pltpu.SemaphoreType.DMA,
          pltpu.SemaphoreType.DMA,
      ],
  )
  return fn(lo, dst2, mvec, zeros32)


def _sc_pass_c(v0, v1a, v1b, v1c, es0f, es1f, eu0f, eu1f, eu2f, src2,
               dst2, zeros32, ep):
  epw = ep // NW
  nchunk = epw // CK
  fpc = CK * CH

  def body(v0_t, va_t, vb_t, vc_t, e0_h, e1_h, u0_h, u1_h, u2_h, src_h,
           dst_h, z_h, agg_o, srcb, dstb, vr, sab, sbb, spagg, semi, semg,
           seml, sems):
    core = lax.axis_index("c")
    sub = lax.axis_index("s")
    base = (core * NSUB + sub) * nchunk
    it = _iota16()

    def one_group(g, v_t, sa_h, sb_h):
      with_u = sb_h is not None
      pltpu.sync_copy(z_h.at[pl.ds(sub * _SUB_SLICE, _SUB_SLICE)],
                      spagg.at[pl.ds(sub * _SUB_SLICE, _SUB_SLICE)])
      plsc.subcore_barrier()

      def issue(ci, buf):
        i1_ = pltpu.async_copy(src_h.at[pl.ds(base + ci, 1)],
                               srcb.at[pl.ds(buf, 1)], semi)
        i2_ = pltpu.async_copy(dst_h.at[pl.ds(base + ci, 1)],
                               dstb.at[pl.ds(buf, 1)], semi)
        pltpu.async_copy(sa_h.at[pl.ds((base + ci) * fpc, fpc)],
                         sab.at[pl.ds(buf * fpc, fpc)], seml)
        if with_u:
          pltpu.async_copy(sb_h.at[pl.ds((base + ci) * fpc, fpc)],
                           sbb.at[pl.ds(buf * fpc, fpc)], seml)
        i1_.wait()
        i2_.wait()
        pltpu.async_copy(v_t.at[srcb.at[buf]], vr.at[pl.ds(buf * CK, CK)],
                         semg)

      def drain_loads(buf):
        pltpu.make_async_copy(sa_h.at[pl.ds(0, fpc)],
                              sab.at[pl.ds(buf * fpc, fpc)], seml).wait()
        if with_u:
          pltpu.make_async_copy(sb_h.at[pl.ds(0, fpc)],
                                sbb.at[pl.ds(buf * fpc, fpc)], seml).wait()
        pltpu.make_async_copy(v_t.at[srcb.at[buf]],
                              vr.at[pl.ds(buf * CK, CK)], semg).wait()

      issue(0, 0)

      def chunk(ci, carry):
        buf = lax.rem(ci, 2)
        nbuf = 1 - buf

        @pl.when(ci > 0)
        def _():
          pltpu.make_async_copy(vr.at[pl.ds(nbuf * CK, CK)],
                                spagg.at[dstb.at[nbuf]], sems).wait()

        @pl.when(ci + 1 < nchunk)
        def _():
          issue(ci + 1, nbuf)

        drain_loads(buf)
        rbase = buf * CK
        fbase = buf * fpc

        def mgrp(jo, carry2):
          gb = jo * 8
          for jj in range(8):
            g2 = gb + jj
            erow = jnp.full((16,), rbase, i32) + _splat(g2 >> 1)
            cols = it + (g2 & 1) * 16
            fidx = it + fbase + g2 * 16
            v16 = plsc.load_gather(vr, [erow, cols])
            s16 = plsc.load_gather(sab, [fidx])
            if with_u:
              s2_16 = plsc.load_gather(sbb, [fidx])
              msg = v16 * s16 + s2_16
            else:
              msg = v16 * s16
            plsc.store_scatter(vr, [erow, cols], msg)
          return carry2

        lax.fori_loop(0, (2 * CK) // 8, mgrp, 0)
        pltpu.async_copy(vr.at[pl.ds(rbase, CK)], spagg.at[dstb.at[buf]],
                         sems, add=True)
        return carry

      lax.fori_loop(0, nchunk, chunk, 0)
      lastbuf = lax.rem(nchunk - 1, 2)
      pltpu.make_async_copy(vr.at[pl.ds(lastbuf * CK, CK)],
                            spagg.at[dstb.at[lastbuf]], sems).wait()
      plsc.subcore_barrier()
      pltpu.sync_copy(spagg.at[pl.ds(sub * _SUB_SLICE, _SUB_SLICE)],
                      agg_o.at[core, g, pl.ds(sub * _SUB_SLICE, _SUB_SLICE)])

    one_group(0, v0_t, e0_h, None)
    one_group(1, va_t, e1_h, u0_h)
    one_group(2, vb_t, e1_h, u1_h)
    one_group(3, vc_t, e1_h, u2_h)

  scratch = [
      pltpu.VMEM((2, CK), i32),
      pltpu.VMEM((2, CK), i32),
      pltpu.VMEM((2 * CK, CH), f32),
      pltpu.VMEM((2 * CK * CH,), f32),
      pltpu.VMEM((2 * CK * CH,), f32),
      pltpu.VMEM_SHARED((NPAD, CH), f32),
      pltpu.SemaphoreType.DMA,
      pltpu.SemaphoreType.DMA,
      pltpu.SemaphoreType.DMA,
      pltpu.SemaphoreType.DMA,
  ]
  fn = pl.kernel(
      body,
      out_type=jax.ShapeDtypeStruct((NCORE, 4, NPAD, CH), f32),
      mesh=_mesh(),
      compiler_params=pltpu.CompilerParams(use_tc_tiling_on_sc=False, needs_layout_passes=False),
      scratch_types=scratch,
  )
  return fn(v0, v1a, v1b, v1c, es0f, es1f, eu0f, eu1f, eu2f, src2, dst2,
            zeros32)


def _tc_prep(pgs, pgd, ep):
  def body(pgs_r, pgd_r, ud_r):
    rel = pgs_r[...] - pgd_r[...]
    d2 = jnp.sum(rel * rel, axis=1, keepdims=True)
    dist = jnp.sqrt(d2)
    u = rel[:, :3] / (dist + 1e-6)
    zpad = jnp.zeros((rel.shape[0], 4), f32)
    ud_r[...] = jnp.concatenate([u, dist, zpad], axis=1)

  return pl.pallas_call(
      body,
      grid=(ep // NBLK,),
      in_specs=[
          pl.BlockSpec((NBLK, 8), lambda i: (i, 0)),
          pl.BlockSpec((NBLK, 8), lambda i: (i, 0)),
      ],
      out_specs=pl.BlockSpec((NBLK, 8), lambda i: (i, 0)),
      out_shape=jax.ShapeDtypeStruct((ep, 8), f32),
  )(pgs, pgd)


def _tc_nodeproj(h0, h1, wq, wk, wv0, wv1, wself):
  def body(h0_r, h1_r, wq_r, wk_r, wv0_r, wv1_r, ws_r, q_r, k_r, v0_r,
           v1a_r, v1b_r, v1c_r, sc_r):
    h0b = h0_r[...]
    q_r[...] = jnp.dot(h0b, wq_r[...]) * SCALE
    k_r[...] = jnp.dot(h0b, wk_r[...])
    v0_r[...] = jnp.dot(h0b, wv0_r[...])
    sc_r[...] = jnp.dot(h0b, ws_r[...])
    h1b = h1_r[...]
    wv1b = wv1_r[...]
    v1a_r[...] = jnp.dot(h1b[0], wv1b)
    v1b_r[...] = jnp.dot(h1b[1], wv1b)
    v1c_r[...] = jnp.dot(h1b[2], wv1b)

  wspec = pl.BlockSpec((CH, CH), lambda i: (0, 0))
  nspec = pl.BlockSpec((NBLK, CH), lambda i: (i, 0))
  tspec = pl.BlockSpec((3, NBLK, CH), lambda i: (0, i, 0))
  return pl.pallas_call(
      body,
      grid=(NPAD // NBLK,),
      in_specs=[nspec, tspec, wspec, wspec, wspec, wspec, wspec],
      out_specs=[nspec] * 3 + [nspec] * 3 + [nspec],
      out_shape=[jax.ShapeDtypeStruct((NPAD, CH), f32)] * 7,
  )(h0, h1, wq, wk, wv0, wv1, wself)


def _tc_radial_premult(ud, ea, ex, wr1d, wr1e, br1, wr2a, wr2b, wr2c, ep):
  def body(ud_r, ea_r, ex_r, w1d_r, w1e_r, b1_r, w2a_r, w2b_r, w2c_r,
           es0_r, es1_r, eu0_r, eu1_r, eu2_r):
    udb = ud_r[...]
    dist = udb[:, 3:4]
    h = jnp.maximum(
        jnp.dot(ea_r[...], w1e_r[...]) + dist * w1d_r[...] + b1_r[...], 0.0)
    exb = ex_r[...][:, None]
    es0_r[...] = exb * jnp.dot(h, w2a_r[...])
    es1_r[...] = exb * jnp.dot(h, w2b_r[...])
    es2 = exb * jnp.dot(h, w2c_r[...])
    eu0_r[...] = udb[:, 0:1] * es2
    eu1_r[...] = udb[:, 1:2] * es2
    eu2_r[...] = udb[:, 2:3] * es2

  espec = pl.BlockSpec((NBLK, CH), lambda i: (i, 0))
  wspec = pl.BlockSpec((CH, CH), lambda i: (0, 0))
  rspec = pl.BlockSpec((1, CH), lambda i: (0, 0))
  eshape = jax.ShapeDtypeStruct((ep, CH), f32)
  return pl.pallas_call(
      body,
      grid=(ep // NBLK,),
      in_specs=[
          pl.BlockSpec((NBLK, 8), lambda i: (i, 0)), espec,
          pl.BlockSpec((NBLK,), lambda i: (i,)), rspec, wspec,
          rspec, wspec, wspec, wspec
      ],
      out_specs=[espec] * 5,
      out_shape=[eshape] * 5,
  )(ud, ea, ex, wr1d, wr1e, br1, wr2a, wr2b, wr2c)


def _tc_combine_max(mpart):
  def body(mp_r, m_r):
    m_r[...] = jnp.max(mp_r[...], axis=0)

  return pl.pallas_call(
      body,
      grid=(NPAD // NBLK,),
      in_specs=[pl.BlockSpec((NW, NBLK), lambda i: (0, i))],
      out_specs=pl.BlockSpec((NBLK,), lambda i: (i,)),
      out_shape=jax.ShapeDtypeStruct((NPAD,), f32),
  )(mpart)


def _tc_update(aggp, selfc, h1, denp):
  def body(ap_r, sc_r, h1_r, dp_r, h0o_r, h1o_r):
    den = dp_r[0, :, 0] + dp_r[1, :, 0]
    rden = 1.0 / (den + 1e-9)
    ap = ap_r[...]
    h0o_r[...] = jnp.maximum((ap[0, 0] + ap[1, 0]) * rden[:, None]
                             + sc_r[...], 0.0)
    a1 = ap[0, 1:4] + ap[1, 1:4]
    h1o_r[...] = h1_r[...] + a1 * rden[None, :, None]

  nspec = pl.BlockSpec((NBLK, CH), lambda i: (i, 0))
  tspec = pl.BlockSpec((3, NBLK, CH), lambda i: (0, i, 0))
  return pl.pallas_call(
      body,
      grid=(NPAD // NBLK,),
      in_specs=[
          pl.BlockSpec((NCORE, 4, NBLK, CH), lambda i: (0, 0, i, 0)),
          nspec,
          tspec,
          pl.BlockSpec((NCORE, NBLK, 8), lambda i: (0, i, 0)),
      ],
      out_specs=[nspec, tspec],
      out_shape=[
          jax.ShapeDtypeStruct((NPAD, CH), f32),
          jax.ShapeDtypeStruct((3, NPAD, CH), f32),
      ],
  )(aggp, selfc, h1, denp)


def _tc_heads(h0, h1, wout0, wout1, wwo, bwo, wwb, wwc1, bwc1, wwc2, wc2):
  def body(h0_r, h1_r, wo0_r, wo1_r, wwo_r, bwo_r, wwb_r, wc1_r, bc1_r,
           wc2_r, wcs_r, out_r):
    hs0 = jnp.dot(h0_r[...], wo0_r[...])
    h1b = h1_r[...]
    hs1 = jnp.einsum("dnc,co->dno", h1b, wo1_r[...],
                     preferred_element_type=f32)
    wo = jnp.tanh(jnp.dot(hs0, wwo_r[...]) + bwo_r[...])
    wb = jnp.tanh(jnp.dot(hs0, wwb_r[...]))
    wcmid = jnp.dot(hs0, wc1_r[...]) + bc1_r[...]
    wc = jnp.maximum(jnp.dot(wcmid, wc2_r[...]), 0.0)
    cs = jnp.dot(hs0, wcs_r[...])
    nb = hs0.shape[0]
    hs1flat = jnp.concatenate([hs1[0], hs1[1], hs1[2]], axis=1)
    out_r[...] = jnp.concatenate(
        [wo, wb, wc, cs, hs1flat, hs0,
         jnp.zeros((nb, 128 - 117), f32)], axis=1)

  nspec = pl.BlockSpec((NBLK, CH), lambda i: (i, 0))
  tspec = pl.BlockSpec((3, NBLK, CH), lambda i: (0, i, 0))

  def w(shape):
    return pl.BlockSpec(shape, lambda i: tuple(0 for _ in shape))

  return pl.pallas_call(
      body,
      grid=(NPAD // NBLK,),
      in_specs=[
          nspec, tspec,
          w((CH, 32)), w((CH, 8)), w((32, 8)), w((1, 8)), w((32, 8)),
          w((32, 32)), w((1, 32)), w((32, 15)), w((32, 30)),
      ],
      out_specs=pl.BlockSpec((NBLK, 128), lambda i: (i, 0)),
      out_shape=jax.ShapeDtypeStruct((NPAD, 128), f32),
  )(h0, h1, wout0, wout1, wwo, bwo, wwb, wwc1, bwc1, wwc2, wc2)


def kernel(x0, edge_index, edge_attr, pos, params):
  p = params
  e_real = edge_index.shape[1]
  ep = ((e_real + NW * CK - 1) // (NW * CK)) * (NW * CK)

  src = jnp.pad(edge_index[0].astype(i32), (0, ep - e_real))
  dst = jnp.pad(edge_index[1].astype(i32), (0, ep - e_real),
                constant_values=NN)
  ea = jnp.pad(edge_attr[:, :, 0], ((0, ep - e_real), (0, 0)))
  pos_pad = jnp.pad(pos, ((0, NPAD - NN), (0, 5)))
  h0 = jnp.pad(x0[:, :, 0], ((0, NPAD - NN), (0, 0)))
  h1 = jnp.zeros((3, NPAD, CH), f32)
  zeros32 = jnp.zeros((NPAD, CH), f32)

  src2 = src.reshape(-1, CK)
  dst2 = dst.reshape(-1, CK)
  pgs, pgd = _sc_posgather(pos_pad, src2, dst2, ep)
  ud = _tc_prep(pgs, pgd, ep)

  for l in range(2):
    q, k, v0, v1a, v1b, v1c, selfc = _tc_nodeproj(
        h0, h1, p["Wq%d" % l], p["Wk%d" % l], p["Wv0%d" % l],
        p["Wv1%d" % l], p["Wself%d" % l])
    lo, mpart = _sc_pass_a(q, k, src2, dst2, ep)
    mvec = _tc_combine_max(mpart)
    ex, denp = _sc_pass_b(lo, dst2, mvec, zeros32, ep)
    wr1 = p["Wr1%d" % l]
    wr2 = p["Wr2%d" % l]
    es0, es1, eu0, eu1, eu2 = _tc_radial_premult(
        ud, ea, ex, wr1[0:1], wr1[1:33], p["br1%d" % l].reshape(1, CH),
        wr2[:, 0:CH], wr2[:, CH:2 * CH], wr2[:, 2 * CH:3 * CH], ep)
    aggp = _sc_pass_c(v0, v1a, v1b, v1c, es0.reshape(-1), es1.reshape(-1),
                      eu0.reshape(-1), eu1.reshape(-1), eu2.reshape(-1),
                      src2, dst2, zeros32, ep)
    h0, h1 = _tc_update(aggp, selfc, h1, denp)

  wc_mat = jnp.transpose(p["Wc"], (1, 0, 2)).reshape(CH, 30)
  pack = _tc_heads(h0, h1, p["Wout0"], p["Wout1"], p["Wwo"],
                   p["bwo"].reshape(1, 8), p["Wwb"], p["Wwc1"],
                   p["bwc1"].reshape(1, CH), p["Wwc2"], wc_mat)

  n = NN
  wo = pack[:n, 0:8]
  wb = pack[:n, 8:16]
  wc = pack[:n, 16:31]
  cs = pack[:n, 31:61].reshape(n, 15, 2).transpose(1, 0, 2)
  hs1 = pack[:n, 61:85].reshape(n, 3, 8).transpose(0, 2, 1)
  hs0 = pack[:n, 85:117]
  return (wo, wb, wc, cs, hs0, hs1)
